# Initial kernel scaffold; baseline (speedup 1.0000x reference)
import jax
import jax.numpy as jnp
from jax.experimental import pallas as pl


def kernel(x, edge_index, W1, b1, W2, b2):
    raise NotImplementedError("write your pallas kernel here")


# speedup vs baseline: 82.4564x; 82.4564x over previous
import jax
import jax.numpy as jnp
from jax import lax
from jax.experimental import pallas as pl
from jax.experimental.pallas import tpu as pltpu
from jax.experimental.pallas import tpu_sc as plsc

N_NODES = 10000
N_EDGES = 320000
D_FEAT = 128
NODE_A = N_NODES - 2
NODE_B = N_NODES - 1

NPAD = 16384
NR = NPAD // 128

CS = 256
E2CAP = 128
E1CAP = 512

_NEG_SLOPE = 0.2


def _iota16():
    return lax.broadcasted_iota(jnp.int32, (16,), 0)


def _store1(ref, idxs, val, dtype):
    lane0 = _iota16() == 0
    vecs = [jnp.full((16,), i, jnp.int32) for i in idxs]
    plsc.store_scatter(ref, vecs, jnp.full((16,), val, dtype), mask=lane0)


def _add1(ref, idxs, val, dtype):
    lane0 = _iota16() == 0
    vecs = [jnp.full((16,), i, jnp.int32) for i in idxs]
    plsc.addupdate_scatter(ref, vecs, jnp.full((16,), val, dtype), mask=lane0)


def _load1(ref, idxs):
    lane0 = _iota16() == 0
    vecs = [jnp.full((16,), i, jnp.int32) for i in idxs]
    return plsc.load_gather(ref, vecs, mask=lane0)[0]


def _rsqrt16(x):
    i = plsc.bitcast(x, jnp.int32)
    i = 0x5F3759DF - lax.shift_right_logical(i, 1)
    y = plsc.bitcast(i, jnp.float32)
    for _ in range(3):
        y = y * (1.5 - 0.5 * x * y * y)
    return y


def _row(v):
    return lax.shift_right_logical(v, 7)


def _col(v):
    return v & 127


def _k1_body(dst_hbm, deg_hbm, dst_v, degloc_v, acc_v, tmp_v, sh_all, sem):
    cid = lax.axis_index("c")
    sid = lax.axis_index("s")
    wid = cid * 16 + sid
    epw = N_EDGES // 32

    def zero_body(j, _):
        for c in range(8):
            degloc_v[j, pl.ds(c * 16, 16)] = jnp.zeros((16,), jnp.float32)
        return 0
    lax.fori_loop(0, NR, zero_body, 0)

    pltpu.sync_copy(dst_hbm.at[pl.ds(wid * epw, epw)], dst_v)

    def hist_body(i, _):
        v = dst_v[pl.ds(i * 16, 16)]
        cnt, lastm = plsc.scan_count(v)
        plsc.addupdate_scatter(
            degloc_v, [_row(v), _col(v)], cnt.astype(jnp.float32), mask=lastm)
        return 0
    lax.fori_loop(0, epw // 16, hist_body, 0)

    pltpu.sync_copy(degloc_v, sh_all.at[sid])
    plsc.subcore_barrier()

    rpt = NR // 16
    base = sid * rpt
    pltpu.sync_copy(sh_all.at[0, pl.ds(base, rpt)], acc_v)
    for k in range(1, 16):
        pltpu.sync_copy(sh_all.at[k, pl.ds(base, rpt)], tmp_v)

        def add_body(j, _):
            for c in range(8):
                acc_v[j, pl.ds(c * 16, 16)] = (
                    acc_v[j, pl.ds(c * 16, 16)] + tmp_v[j, pl.ds(c * 16, 16)])
            return 0
        lax.fori_loop(0, rpt, add_body, 0)

    pltpu.sync_copy(acc_v, deg_hbm.at[cid, pl.ds(base, rpt)])


def _make_k1():
    mesh = plsc.VectorSubcoreMesh(core_axis_name="c", subcore_axis_name="s")
    return pl.kernel(
        _k1_body,
        out_type=jax.ShapeDtypeStruct((2, NR, 128), jnp.float32),
        mesh=mesh,
        compiler_params=pltpu.CompilerParams(needs_layout_passes=False),
        scratch_types=[
            pltpu.VMEM((N_EDGES // 32,), jnp.int32),
            pltpu.VMEM((NR, 128), jnp.float32),
            pltpu.VMEM((NR // 16, 128), jnp.float32),
            pltpu.VMEM((NR // 16, 128), jnp.float32),
            pltpu.VMEM_SHARED((16, NR, 128), jnp.float32),
            pltpu.SemaphoreType.DMA,
        ],
    )


def _k2_body(src_hbm, dst_hbm, degp_hbm, x_hbm,
             agg1_hbm, amat_hbm,
             src_v, dst_v, dinv_v, table_v,
             tmpa_v, tmpb_v,
             e2src_v, e2dst_v, cnt16_v,
             e1src_v, e1dst_v, e1slot_v, e1w_v,
             slotnodes_v, amat_v,
             e2a_src_v, e2a_dst_v, e2a_cnt_v,
             chunk_src_v, chunk_slot_v, rows_v, tmp16_v, agg1loc_v,
             sh_dinv, sh_table, sh_slotnodes, sh_cnt,
             sh_e2src, sh_e2dst, sh_e2cnt, sh_slab,
             sem):
    sid = lax.axis_index("s")
    epw = N_EDGES // 16
    rpt = NR // 16

    base = sid * rpt
    pltpu.sync_copy(degp_hbm.at[0, pl.ds(base, rpt)], tmpa_v)
    pltpu.sync_copy(degp_hbm.at[1, pl.ds(base, rpt)], tmpb_v)

    def dinv_body(j, _):
        for c in range(8):
            deg = (tmpa_v[j, pl.ds(c * 16, 16)]
                   + tmpb_v[j, pl.ds(c * 16, 16)] + 1.0)
            tmpa_v[j, pl.ds(c * 16, 16)] = _rsqrt16(deg)
        return 0
    lax.fori_loop(0, rpt, dinv_body, 0)
    pltpu.sync_copy(tmpa_v, sh_dinv.at[pl.ds(base, rpt)])

    def zrow_body(j, _):
        for c in range(8):
            agg1loc_v[j, pl.ds(c * 16, 16)] = jnp.zeros((16,), jnp.float32)
        return 0
    lax.fori_loop(0, CS, zrow_body, 0)

    pltpu.sync_copy(dst_hbm.at[pl.ds(sid * epw, epw)], dst_v)
    pltpu.sync_copy(src_hbm.at[pl.ds(sid * epw, epw)], src_v)

    plsc.subcore_barrier()
    pltpu.sync_copy(sh_dinv, dinv_v)

    def e2_body(i, off):
        v = dst_v[pl.ds(i * 16, 16)]
        m = v >= NODE_A
        sv = src_v[pl.ds(i * 16, 16)]
        plsc.store_compressed(e2src_v.at[pl.ds(off, 16)], sv, mask=m)
        plsc.store_compressed(e2dst_v.at[pl.ds(off, 16)], v, mask=m)
        off = off + jnp.sum(m.astype(jnp.int32))
        return jnp.minimum(off, E2CAP - 16)
    off2 = lax.fori_loop(0, epw // 16, e2_body, jnp.int32(0))

    cnt16_v[...] = jnp.full((16,), off2, jnp.int32)
    pltpu.sync_copy(e2src_v, sh_e2src.at[sid])
    pltpu.sync_copy(e2dst_v, sh_e2dst.at[sid])
    pltpu.sync_copy(cnt16_v, sh_e2cnt.at[sid])
    plsc.subcore_barrier()

    @pl.when(sid == 0)
    def _dedup():
        def tneg_body(j, _):
            for c in range(8):
                table_v[j, pl.ds(c * 16, 16)] = jnp.full((16,), -1, jnp.int32)
            return 0
        lax.fori_loop(0, NR, tneg_body, 0)

        def sn_body(j, _):
            slotnodes_v[pl.ds(j * 16, 16)] = jnp.zeros((16,), jnp.int32)
            return 0
        lax.fori_loop(0, CS // 16, sn_body, 0)

        def az_body(j, _):
            for r in range(8):
                amat_v[r, pl.ds(j * 16, 16)] = jnp.zeros((16,), jnp.float32)
            return 0
        lax.fori_loop(0, CS // 16, az_body, 0)

        pltpu.sync_copy(sh_e2src, e2a_src_v)
        pltpu.sync_copy(sh_e2dst, e2a_dst_v)
        pltpu.sync_copy(sh_e2cnt, e2a_cnt_v)

        _store1(table_v, [NODE_A >> 7, NODE_A & 127], jnp.int32(0), jnp.int32)
        _store1(table_v, [NODE_B >> 7, NODE_B & 127], jnp.int32(1), jnp.int32)
        _store1(slotnodes_v, [0], jnp.int32(NODE_A), jnp.int32)
        _store1(slotnodes_v, [1], jnp.int32(NODE_B), jnp.int32)
        dA = _load1(dinv_v, [NODE_A >> 7, NODE_A & 127])
        dB = _load1(dinv_v, [NODE_B >> 7, NODE_B & 127])
        _store1(amat_v, [0, 0], dA * dA, jnp.float32)
        _store1(amat_v, [1, 1], dB * dB, jnp.float32)

        cnt = jnp.int32(2)
        for t in range(16):
            ct = e2a_cnt_v[t, pl.ds(0, 16)][0]

            def e2e_body(i, cnt):
                s = _load1(e2a_src_v, [t, i])
                d = _load1(e2a_dst_v, [t, i])
                sl = _load1(table_v, [_row(s), _col(s)])
                isnew = sl < 0
                slot = jnp.where(isnew, cnt, sl)
                slot = jnp.minimum(slot, CS - 1)
                _store1(table_v, [_row(s), _col(s)], slot, jnp.int32)
                _store1(slotnodes_v, [slot], s, jnp.int32)
                w = (_load1(dinv_v, [_row(s), _col(s)])
                     * _load1(dinv_v, [_row(d), _col(d)]))
                r = d - NODE_A
                _add1(amat_v, [r, slot], w, jnp.float32)
                return cnt + isnew.astype(jnp.int32)
            cnt = lax.fori_loop(0, ct, e2e_body, cnt)

        cnt = jnp.minimum(cnt, CS)
        pltpu.sync_copy(amat_v, amat_hbm)
        pltpu.sync_copy(table_v, sh_table)
        pltpu.sync_copy(slotnodes_v, sh_slotnodes)
        cnt16_v[...] = jnp.full((16,), cnt, jnp.int32)
        pltpu.sync_copy(cnt16_v, sh_cnt)

    plsc.subcore_barrier()

    pltpu.sync_copy(sh_table, table_v)
    pltpu.sync_copy(sh_slotnodes, slotnodes_v)
    pltpu.sync_copy(sh_cnt, cnt16_v)
    cnt = cnt16_v[...][0]

    def ez_body(j, _):
        z = jnp.zeros((16,), jnp.int32)
        e1src_v[pl.ds(j * 16, 16)] = z
        e1dst_v[pl.ds(j * 16, 16)] = z
        e1slot_v[pl.ds(j * 16, 16)] = z
        return 0
    lax.fori_loop(0, E1CAP // 16, ez_body, 0)

    def e1_body(i, off):
        v = dst_v[pl.ds(i * 16, 16)]
        tv = plsc.load_gather(table_v, [_row(v), _col(v)])
        m = tv >= 0
        sv = src_v[pl.ds(i * 16, 16)]
        plsc.store_compressed(e1src_v.at[pl.ds(off, 16)], sv, mask=m)
        plsc.store_compressed(e1dst_v.at[pl.ds(off, 16)], v, mask=m)
        plsc.store_compressed(e1slot_v.at[pl.ds(off, 16)], tv, mask=m)
        off = off + jnp.sum(m.astype(jnp.int32))
        return jnp.minimum(off, E1CAP - 16)
    off1 = lax.fori_loop(0, epw // 16, e1_body, jnp.int32(0))

    def self_cond(state):
        j, _ = state
        return j < cnt

    def self_body(state):
        j, off = state
        n = _load1(slotnodes_v, [j])
        _store1(e1src_v, [off], n, jnp.int32)
        _store1(e1dst_v, [off], n, jnp.int32)
        _store1(e1slot_v, [off], j, jnp.int32)
        return (j + 16, jnp.minimum(off + 1, E1CAP - 1))
    _, off1 = lax.while_loop(self_cond, self_body, (sid, off1))
    nloc = off1

    def w_body(k, _):
        s = e1src_v[pl.ds(k * 16, 16)]
        d = e1dst_v[pl.ds(k * 16, 16)]
        wv = (plsc.load_gather(dinv_v, [_row(s), _col(s)])
              * plsc.load_gather(dinv_v, [_row(d), _col(d)]))
        lane = k * 16 + _iota16()
        e1w_v[pl.ds(k * 16, 16)] = jnp.where(lane < nloc, wv, 0.0)
        return 0
    lax.fori_loop(0, E1CAP // 16, w_body, 0)

    nch = lax.shift_right_logical(nloc + 15, 4)

    def chunk_body(k, _):
        @pl.when(k < nch)
        def _do():
            chunk_src_v[...] = e1src_v[pl.ds(k * 16, 16)]
            chunk_slot_v[...] = e1slot_v[pl.ds(k * 16, 16)]
            pltpu.async_copy(x_hbm.at[chunk_src_v], rows_v, sem).wait()
            wchunk = e1w_v[pl.ds(k * 16, 16)]
            slotchunk = chunk_slot_v[...]
            for r in range(16):
                wb = jnp.full((16,), wchunk[r], jnp.float32)
                slot = slotchunk[r]
                for c in range(8):
                    agg1loc_v[slot, pl.ds(c * 16, 16)] = (
                        agg1loc_v[slot, pl.ds(c * 16, 16)]
                        + rows_v[r, pl.ds(c * 16, 16)] * wb)
        return 0
    lax.fori_loop(0, E1CAP // 16, chunk_body, 0)

    spt = CS // 16
    rb = sid * spt

    def zrow16(j, _):
        for c in range(8):
            rows_v[j, pl.ds(c * 16, 16)] = jnp.zeros((16,), jnp.float32)
        return 0
    lax.fori_loop(0, spt, zrow16, 0)

    for g in range(16):
        owner = lax.rem(sid + g, 16)
        obase = pl.multiple_of(owner * spt, spt)
        pltpu.sync_copy(agg1loc_v.at[pl.ds(obase, spt)], sh_slab.at[sid])
        plsc.subcore_barrier()
        srcslot = lax.rem(sid - g + 16, 16)
        pltpu.sync_copy(sh_slab.at[srcslot], tmp16_v)

        def red_body(j, _):
            for c in range(8):
                rows_v[j, pl.ds(c * 16, 16)] = (
                    rows_v[j, pl.ds(c * 16, 16)]
                    + tmp16_v[j, pl.ds(c * 16, 16)])
            return 0
        lax.fori_loop(0, spt, red_body, 0)
        plsc.subcore_barrier()

    pltpu.sync_copy(rows_v, agg1_hbm.at[pl.ds(rb, spt)])


def _make_k2():
    mesh = plsc.VectorSubcoreMesh(core_axis_name="c", subcore_axis_name="s",
                                  num_cores=1)
    epw = N_EDGES // 16
    return pl.kernel(
        _k2_body,
        out_type=(
            jax.ShapeDtypeStruct((CS, D_FEAT), jnp.float32),
            jax.ShapeDtypeStruct((8, CS), jnp.float32),
        ),
        mesh=mesh,
        compiler_params=pltpu.CompilerParams(needs_layout_passes=False),
        scratch_types=[
            pltpu.VMEM((epw,), jnp.int32),
            pltpu.VMEM((epw,), jnp.int32),
            pltpu.VMEM((NR, 128), jnp.float32),
            pltpu.VMEM((NR, 128), jnp.int32),
            pltpu.VMEM((NR // 16, 128), jnp.float32),
            pltpu.VMEM((NR // 16, 128), jnp.float32),
            pltpu.VMEM((E2CAP,), jnp.int32),
            pltpu.VMEM((E2CAP,), jnp.int32),
            pltpu.VMEM((16,), jnp.int32),
            pltpu.VMEM((E1CAP,), jnp.int32),
            pltpu.VMEM((E1CAP,), jnp.int32),
            pltpu.VMEM((E1CAP,), jnp.int32),
            pltpu.VMEM((E1CAP,), jnp.float32),
            pltpu.VMEM((CS,), jnp.int32),
            pltpu.VMEM((8, CS), jnp.float32),
            pltpu.VMEM((16, E2CAP), jnp.int32),
            pltpu.VMEM((16, E2CAP), jnp.int32),
            pltpu.VMEM((16, 16), jnp.int32),
            pltpu.VMEM((16,), jnp.int32),
            pltpu.VMEM((16,), jnp.int32),
            pltpu.VMEM((16, D_FEAT), jnp.float32),
            pltpu.VMEM((16, D_FEAT), jnp.float32),
            pltpu.VMEM((CS, D_FEAT), jnp.float32),
            pltpu.VMEM_SHARED((NR, 128), jnp.float32),
            pltpu.VMEM_SHARED((NR, 128), jnp.int32),
            pltpu.VMEM_SHARED((CS,), jnp.int32),
            pltpu.VMEM_SHARED((16,), jnp.int32),
            pltpu.VMEM_SHARED((16, E2CAP), jnp.int32),
            pltpu.VMEM_SHARED((16, E2CAP), jnp.int32),
            pltpu.VMEM_SHARED((16, 16), jnp.int32),
            pltpu.VMEM_SHARED((16, CS // 16, D_FEAT), jnp.float32),
            pltpu.SemaphoreType.DMA,
        ],
    )


def _k3_body(agg1_ref, w1_ref, b1_ref, amat_ref, w2_ref, b2_ref, out_ref):
    h1 = jnp.dot(agg1_ref[...], w1_ref[...],
                 preferred_element_type=jnp.float32) + b1_ref[...]
    act = jnp.where(h1 >= 0, h1, h1 * _NEG_SLOPE)
    t = jnp.dot(amat_ref[...], act, preferred_element_type=jnp.float32)
    h2 = jnp.dot(t, w2_ref[...],
                 preferred_element_type=jnp.float32) + b2_ref[...]
    col = lax.broadcasted_iota(jnp.int32, (1, 128), 1)
    valid = col < 16
    rowb = h2[1:2, :]
    m = jnp.max(jnp.where(valid, rowb, -jnp.inf))
    e = jnp.where(valid, jnp.exp(rowb - m), 0.0)
    f1 = e / jnp.sum(e)
    rowa = h2[0:1, :]
    out_ref[...] = jnp.concatenate(
        [f1, rowa, jnp.zeros((6, 128), jnp.float32)], axis=0)


def _k3(agg1, w1p, b1p, amat, w2p, b2p):
    return pl.pallas_call(
        _k3_body,
        out_shape=jax.ShapeDtypeStruct((8, 128), jnp.float32),
    )(agg1, w1p, b1p, amat, w2p, b2p)


@jax.jit
def kernel(x, edge_index, W1, b1, W2, b2):
    src = edge_index[0].astype(jnp.int32)
    dst = edge_index[1].astype(jnp.int32)

    deg_parts = _make_k1()(dst)
    agg1, amat = _make_k2()(src, dst, deg_parts, x)

    hidden = W1.shape[1]
    w1p = jnp.pad(W1, ((0, 0), (0, 256 - hidden)))
    b1p = jnp.pad(b1, (0, 256 - hidden)).reshape(1, 256)
    w2p = jnp.pad(W2, ((0, 256 - hidden), (0, 128 - W2.shape[1])))
    b2p = jnp.pad(b2, (0, 128 - W2.shape[1])).reshape(1, 128)

    out = _k3(agg1, w1p, b1p, amat, w2p, b2p)
    return jnp.concatenate([out[0, :16], out[1, :16]], axis=0)


# speedup vs baseline: 82.4938x; 1.0005x over previous
import jax
import jax.numpy as jnp
from jax import lax
from jax.experimental import pallas as pl
from jax.experimental.pallas import tpu as pltpu
from jax.experimental.pallas import tpu_sc as plsc

N_NODES = 10000
N_EDGES = 320000
D_FEAT = 128
NODE_A = N_NODES - 2
NODE_B = N_NODES - 1

NPAD = 16384
NR = NPAD // 128

CS = 256
E2CAP = 64
E1CAP = 512

_NEG_SLOPE = 0.2


def _iota16():
    return lax.broadcasted_iota(jnp.int32, (16,), 0)


def _store1(ref, idxs, val, dtype):
    lane0 = _iota16() == 0
    vecs = [jnp.full((16,), i, jnp.int32) for i in idxs]
    plsc.store_scatter(ref, vecs, jnp.full((16,), val, dtype), mask=lane0)


def _add1(ref, idxs, val, dtype):
    lane0 = _iota16() == 0
    vecs = [jnp.full((16,), i, jnp.int32) for i in idxs]
    plsc.addupdate_scatter(ref, vecs, jnp.full((16,), val, dtype), mask=lane0)


def _load1(ref, idxs):
    lane0 = _iota16() == 0
    vecs = [jnp.full((16,), i, jnp.int32) for i in idxs]
    return plsc.load_gather(ref, vecs, mask=lane0)[0]


def _row(v):
    return lax.shift_right_logical(v, 7)


def _col(v):
    return v & 127


def _k1_body(dst_hbm, src_hbm, deg_hbm, e2s_hbm, e2d_hbm, e2c_hbm,
             dst_v, src_v, degloc_v, acc_v, tmp_v, e2src_v, e2dst_v, cnt16_v,
             sh_all, sem):
    cid = lax.axis_index("c")
    sid = lax.axis_index("s")
    wid = cid * 16 + sid
    epw = N_EDGES // 32

    def zero_body(j, _):
        for c in range(8):
            degloc_v[j, pl.ds(c * 16, 16)] = jnp.zeros((16,), jnp.float32)
        return 0
    lax.fori_loop(0, NR, zero_body, 0)

    pltpu.sync_copy(dst_hbm.at[pl.ds(wid * epw, epw)], dst_v)
    pltpu.sync_copy(src_hbm.at[pl.ds(wid * epw, epw)], src_v)

    def hist_body(i, _):
        v = dst_v[pl.ds(i * 16, 16)]
        cnt, lastm = plsc.scan_count(v)
        plsc.addupdate_scatter(
            degloc_v, [_row(v), _col(v)], cnt.astype(jnp.float32), mask=lastm)
        return 0
    lax.fori_loop(0, epw // 16, hist_body, 0)

    def e2_body(i, off):
        v = dst_v[pl.ds(i * 16, 16)]
        m = v >= NODE_A
        sv = src_v[pl.ds(i * 16, 16)]
        plsc.store_compressed(e2src_v.at[pl.ds(off, 16)], sv, mask=m)
        plsc.store_compressed(e2dst_v.at[pl.ds(off, 16)], v, mask=m)
        off = off + jnp.sum(m.astype(jnp.int32))
        return jnp.minimum(off, E2CAP - 16)
    off2 = lax.fori_loop(0, epw // 16, e2_body, jnp.int32(0))
    cnt16_v[...] = jnp.full((16,), off2, jnp.int32)
    pltpu.sync_copy(e2src_v, e2s_hbm.at[wid])
    pltpu.sync_copy(e2dst_v, e2d_hbm.at[wid])
    pltpu.sync_copy(cnt16_v, e2c_hbm.at[wid])

    pltpu.sync_copy(degloc_v, sh_all.at[sid])
    plsc.subcore_barrier()

    rpt = NR // 16
    base = sid * rpt
    pltpu.sync_copy(sh_all.at[0, pl.ds(base, rpt)], acc_v)
    for k in range(1, 16):
        pltpu.sync_copy(sh_all.at[k, pl.ds(base, rpt)], tmp_v)

        def add_body(j, _):
            for c in range(8):
                acc_v[j, pl.ds(c * 16, 16)] = (
                    acc_v[j, pl.ds(c * 16, 16)] + tmp_v[j, pl.ds(c * 16, 16)])
            return 0
        lax.fori_loop(0, rpt, add_body, 0)

    pltpu.sync_copy(acc_v, deg_hbm.at[cid, pl.ds(base, rpt)])


def _make_k1():
    mesh = plsc.VectorSubcoreMesh(core_axis_name="c", subcore_axis_name="s")
    return pl.kernel(
        _k1_body,
        out_type=(
            jax.ShapeDtypeStruct((2, NR, 128), jnp.float32),
            jax.ShapeDtypeStruct((32, E2CAP), jnp.int32),
            jax.ShapeDtypeStruct((32, E2CAP), jnp.int32),
            jax.ShapeDtypeStruct((32, 16), jnp.int32),
        ),
        mesh=mesh,
        compiler_params=pltpu.CompilerParams(needs_layout_passes=False),
        scratch_types=[
            pltpu.VMEM((N_EDGES // 32,), jnp.int32),
            pltpu.VMEM((N_EDGES // 32,), jnp.int32),
            pltpu.VMEM((NR, 128), jnp.float32),
            pltpu.VMEM((NR // 16, 128), jnp.float32),
            pltpu.VMEM((NR // 16, 128), jnp.float32),
            pltpu.VMEM((E2CAP,), jnp.int32),
            pltpu.VMEM((E2CAP,), jnp.int32),
            pltpu.VMEM((16,), jnp.int32),
            pltpu.VMEM_SHARED((16, NR, 128), jnp.float32),
            pltpu.SemaphoreType.DMA,
        ],
    )


def _k2_body(src_hbm, dst_hbm, dinv_hbm, e2s_hbm, e2d_hbm, e2c_hbm, x_hbm,
             agg1_hbm, amat_hbm,
             src_v, dst_v, dinv_v, table_v,
             cnt16_v,
             e1src_v, e1dst_v, e1slot_v, e1w_v,
             slotnodes_v, amat_v,
             e2a_src_v, e2a_dst_v, e2a_cnt_v,
             chunk_src_v, chunk_slot_v, rows_v, tmp16_v, agg1loc_v,
             sh_slotnodes, sh_cnt,
             sh_slab,
             sem):
    sid = lax.axis_index("s")
    epw = N_EDGES // 16

    pltpu.sync_copy(dinv_hbm, dinv_v)

    def zrow_body(j, _):
        for c in range(8):
            agg1loc_v[j, pl.ds(c * 16, 16)] = jnp.zeros((16,), jnp.float32)
        return 0
    lax.fori_loop(0, CS, zrow_body, 0)

    pltpu.sync_copy(dst_hbm.at[pl.ds(sid * epw, epw)], dst_v)
    pltpu.sync_copy(src_hbm.at[pl.ds(sid * epw, epw)], src_v)

    @pl.when(sid == 0)
    def _dedup():
        def tneg_body(j, _):
            for c in range(8):
                table_v[j, pl.ds(c * 16, 16)] = jnp.full((16,), -1, jnp.int32)
            return 0
        lax.fori_loop(0, NR, tneg_body, 0)

        def sn_body(j, _):
            slotnodes_v[pl.ds(j * 16, 16)] = jnp.zeros((16,), jnp.int32)
            return 0
        lax.fori_loop(0, CS // 16, sn_body, 0)

        def az_body(j, _):
            for r in range(8):
                amat_v[r, pl.ds(j * 16, 16)] = jnp.zeros((16,), jnp.float32)
            return 0
        lax.fori_loop(0, CS // 16, az_body, 0)

        pltpu.sync_copy(e2s_hbm, e2a_src_v)
        pltpu.sync_copy(e2d_hbm, e2a_dst_v)
        pltpu.sync_copy(e2c_hbm, e2a_cnt_v)

        _store1(table_v, [NODE_A >> 7, NODE_A & 127], jnp.int32(0), jnp.int32)
        _store1(table_v, [NODE_B >> 7, NODE_B & 127], jnp.int32(1), jnp.int32)
        _store1(slotnodes_v, [0], jnp.int32(NODE_A), jnp.int32)
        _store1(slotnodes_v, [1], jnp.int32(NODE_B), jnp.int32)
        dA = _load1(dinv_v, [NODE_A >> 7, NODE_A & 127])
        dB = _load1(dinv_v, [NODE_B >> 7, NODE_B & 127])
        _store1(amat_v, [0, 0], dA * dA, jnp.float32)
        _store1(amat_v, [1, 1], dB * dB, jnp.float32)

        cnt = jnp.int32(2)
        for t in range(32):
            ct = e2a_cnt_v[t, pl.ds(0, 16)][0]

            def e2e_body(i, cnt):
                s = _load1(e2a_src_v, [t, i])
                d = _load1(e2a_dst_v, [t, i])
                sl = _load1(table_v, [_row(s), _col(s)])
                isnew = sl < 0
                slot = jnp.where(isnew, cnt, sl)
                slot = jnp.minimum(slot, CS - 1)
                _store1(table_v, [_row(s), _col(s)], slot, jnp.int32)
                _store1(slotnodes_v, [slot], s, jnp.int32)
                w = (_load1(dinv_v, [_row(s), _col(s)])
                     * _load1(dinv_v, [_row(d), _col(d)]))
                r = d - NODE_A
                _add1(amat_v, [r, slot], w, jnp.float32)
                return cnt + isnew.astype(jnp.int32)
            cnt = lax.fori_loop(0, ct, e2e_body, cnt)

        cnt = jnp.minimum(cnt, CS)
        pltpu.sync_copy(amat_v, amat_hbm)
        pltpu.sync_copy(slotnodes_v, sh_slotnodes)
        cnt16_v[...] = jnp.full((16,), cnt, jnp.int32)
        pltpu.sync_copy(cnt16_v, sh_cnt)

    plsc.subcore_barrier()

    pltpu.sync_copy(sh_slotnodes, slotnodes_v)
    pltpu.sync_copy(sh_cnt, cnt16_v)
    cnt = cnt16_v[...][0]

    @pl.when(sid != 0)
    def _rebuild():
        def tneg_body(j, _):
            for c in range(8):
                table_v[j, pl.ds(c * 16, 16)] = jnp.full((16,), -1, jnp.int32)
            return 0
        lax.fori_loop(0, NR, tneg_body, 0)

        def tb_cond(j):
            return j < cnt

        def tb_body(j):
            n = _load1(slotnodes_v, [j])
            _store1(table_v, [_row(n), _col(n)], j, jnp.int32)
            return j + 1
        lax.while_loop(tb_cond, tb_body, jnp.int32(0))

    def ez_body(j, _):
        z = jnp.zeros((16,), jnp.int32)
        e1src_v[pl.ds(j * 16, 16)] = z
        e1dst_v[pl.ds(j * 16, 16)] = z
        e1slot_v[pl.ds(j * 16, 16)] = z
        return 0
    lax.fori_loop(0, E1CAP // 16, ez_body, 0)

    def e1_body(i, off):
        for u in range(2):
            v = dst_v[pl.ds((2 * i + u) * 16, 16)]
            tv = plsc.load_gather(table_v, [_row(v), _col(v)])
            m = tv >= 0
            sv = src_v[pl.ds((2 * i + u) * 16, 16)]
            plsc.store_compressed(e1src_v.at[pl.ds(off, 16)], sv, mask=m)
            plsc.store_compressed(e1dst_v.at[pl.ds(off, 16)], v, mask=m)
            plsc.store_compressed(e1slot_v.at[pl.ds(off, 16)], tv, mask=m)
            off = off + jnp.sum(m.astype(jnp.int32))
            off = jnp.minimum(off, E1CAP - 16)
        return off
    off1 = lax.fori_loop(0, epw // 32, e1_body, jnp.int32(0))

    def self_cond(state):
        j, _ = state
        return j < cnt

    def self_body(state):
        j, off = state
        n = _load1(slotnodes_v, [j])
        _store1(e1src_v, [off], n, jnp.int32)
        _store1(e1dst_v, [off], n, jnp.int32)
        _store1(e1slot_v, [off], j, jnp.int32)
        return (j + 16, jnp.minimum(off + 1, E1CAP - 1))
    _, off1 = lax.while_loop(self_cond, self_body, (sid, off1))
    nloc = off1

    def w_body(k, _):
        s = e1src_v[pl.ds(k * 16, 16)]
        d = e1dst_v[pl.ds(k * 16, 16)]
        wv = (plsc.load_gather(dinv_v, [_row(s), _col(s)])
              * plsc.load_gather(dinv_v, [_row(d), _col(d)]))
        lane = k * 16 + _iota16()
        e1w_v[pl.ds(k * 16, 16)] = jnp.where(lane < nloc, wv, 0.0)
        return 0
    lax.fori_loop(0, E1CAP // 16, w_body, 0)

    nch = lax.shift_right_logical(nloc + 15, 4)

    def chunk_body(k, _):
        @pl.when(k < nch)
        def _do():
            chunk_src_v[...] = e1src_v[pl.ds(k * 16, 16)]
            chunk_slot_v[...] = e1slot_v[pl.ds(k * 16, 16)]
            pltpu.async_copy(x_hbm.at[chunk_src_v], rows_v, sem).wait()
            wchunk = e1w_v[pl.ds(k * 16, 16)]
            slotchunk = chunk_slot_v[...]
            for r in range(16):
                wb = jnp.full((16,), wchunk[r], jnp.float32)
                slot = slotchunk[r]
                for c in range(8):
                    agg1loc_v[slot, pl.ds(c * 16, 16)] = (
                        agg1loc_v[slot, pl.ds(c * 16, 16)]
                        + rows_v[r, pl.ds(c * 16, 16)] * wb)
        return 0
    lax.fori_loop(0, E1CAP // 16, chunk_body, 0)

    spt = CS // 16
    rb = sid * spt

    def zrow16(j, _):
        for c in range(8):
            rows_v[j, pl.ds(c * 16, 16)] = jnp.zeros((16,), jnp.float32)
        return 0
    lax.fori_loop(0, spt, zrow16, 0)

    for g in range(16):
        owner = lax.rem(sid + g, 16)
        obase = pl.multiple_of(owner * spt, spt)

        @pl.when(obase < cnt)
        def _pub():
            pltpu.sync_copy(agg1loc_v.at[pl.ds(obase, spt)], sh_slab.at[sid])
        plsc.subcore_barrier()

        @pl.when(rb < cnt)
        def _red():
            srcslot = lax.rem(sid - g + 16, 16)
            pltpu.sync_copy(sh_slab.at[srcslot], tmp16_v)

            def red_body(j, _):
                for c in range(8):
                    rows_v[j, pl.ds(c * 16, 16)] = (
                        rows_v[j, pl.ds(c * 16, 16)]
                        + tmp16_v[j, pl.ds(c * 16, 16)])
                return 0
            lax.fori_loop(0, spt, red_body, 0)
        plsc.subcore_barrier()

    pltpu.sync_copy(rows_v, agg1_hbm.at[pl.ds(rb, spt)])


def _make_k2():
    mesh = plsc.VectorSubcoreMesh(core_axis_name="c", subcore_axis_name="s",
                                  num_cores=1)
    epw = N_EDGES // 16
    return pl.kernel(
        _k2_body,
        out_type=(
            jax.ShapeDtypeStruct((CS, D_FEAT), jnp.float32),
            jax.ShapeDtypeStruct((8, CS), jnp.float32),
        ),
        mesh=mesh,
        compiler_params=pltpu.CompilerParams(needs_layout_passes=False),
        scratch_types=[
            pltpu.VMEM((epw,), jnp.int32),
            pltpu.VMEM((epw,), jnp.int32),
            pltpu.VMEM((NR, 128), jnp.float32),
            pltpu.VMEM((NR, 128), jnp.int32),
            pltpu.VMEM((16,), jnp.int32),
            pltpu.VMEM((E1CAP,), jnp.int32),
            pltpu.VMEM((E1CAP,), jnp.int32),
            pltpu.VMEM((E1CAP,), jnp.int32),
            pltpu.VMEM((E1CAP,), jnp.float32),
            pltpu.VMEM((CS,), jnp.int32),
            pltpu.VMEM((8, CS), jnp.float32),
            pltpu.VMEM((32, E2CAP), jnp.int32),
            pltpu.VMEM((32, E2CAP), jnp.int32),
            pltpu.VMEM((32, 16), jnp.int32),
            pltpu.VMEM((16,), jnp.int32),
            pltpu.VMEM((16,), jnp.int32),
            pltpu.VMEM((16, D_FEAT), jnp.float32),
            pltpu.VMEM((16, D_FEAT), jnp.float32),
            pltpu.VMEM((CS, D_FEAT), jnp.float32),
            pltpu.VMEM_SHARED((CS,), jnp.int32),
            pltpu.VMEM_SHARED((16,), jnp.int32),
            pltpu.VMEM_SHARED((16, CS // 16, D_FEAT), jnp.float32),
            pltpu.SemaphoreType.DMA,
        ],
    )


def _k1b_body(degp_ref, dinv_ref):
    d = degp_ref[0] + degp_ref[1] + 1.0
    dinv_ref[...] = lax.rsqrt(d)


def _k1b(deg_parts):
    return pl.pallas_call(
        _k1b_body,
        out_shape=jax.ShapeDtypeStruct((NR, 128), jnp.float32),
    )(deg_parts)


def _k3_body(agg1_ref, w1_ref, b1_ref, amat_ref, w2_ref, b2_ref, out_ref):
    h1 = jnp.dot(agg1_ref[...], w1_ref[...],
                 preferred_element_type=jnp.float32) + b1_ref[...]
    act = jnp.where(h1 >= 0, h1, h1 * _NEG_SLOPE)
    t = jnp.dot(amat_ref[...], act, preferred_element_type=jnp.float32)
    h2 = jnp.dot(t, w2_ref[...],
                 preferred_element_type=jnp.float32) + b2_ref[...]
    col = lax.broadcasted_iota(jnp.int32, (1, 128), 1)
    valid = col < 16
    rowb = h2[1:2, :]
    m = jnp.max(jnp.where(valid, rowb, -jnp.inf))
    e = jnp.where(valid, jnp.exp(rowb - m), 0.0)
    f1 = e / jnp.sum(e)
    rowa = h2[0:1, :]
    out_ref[...] = jnp.concatenate(
        [f1, rowa, jnp.zeros((6, 128), jnp.float32)], axis=0)


def _k3(agg1, w1p, b1p, amat, w2p, b2p):
    return pl.pallas_call(
        _k3_body,
        out_shape=jax.ShapeDtypeStruct((8, 128), jnp.float32),
    )(agg1, w1p, b1p, amat, w2p, b2p)


@jax.jit
def kernel(x, edge_index, W1, b1, W2, b2):
    src = edge_index[0].astype(jnp.int32)
    dst = edge_index[1].astype(jnp.int32)

    deg_parts, e2s, e2d, e2c = _make_k1()(dst, src)
    dinv = _k1b(deg_parts)
    agg1, amat = _make_k2()(src, dst, dinv, e2s, e2d, e2c, x)

    hidden = W1.shape[1]
    w1p = jnp.pad(W1, ((0, 0), (0, 256 - hidden)))
    b1p = jnp.pad(b1, (0, 256 - hidden)).reshape(1, 256)
    w2p = jnp.pad(W2, ((0, 256 - hidden), (0, 128 - W2.shape[1])))
    b2p = jnp.pad(b2, (0, 128 - W2.shape[1])).reshape(1, 128)

    out = _k3(agg1, w1p, b1p, amat, w2p, b2p)
    return jnp.concatenate([out[0, :16], out[1, :16]], axis=0)


# speedup vs baseline: 87.3317x; 1.0586x over previous
import jax
import jax.numpy as jnp
from jax import lax
from jax.experimental import pallas as pl
from jax.experimental.pallas import tpu as pltpu
from jax.experimental.pallas import tpu_sc as plsc

N_NODES = 10000
N_EDGES = 320000
D_FEAT = 128
NODE_A = N_NODES - 2
NODE_B = N_NODES - 1

NPAD = 16384
NR = NPAD // 128

CS = 256
E2CAP = 64
E1CAP = 512

_NEG_SLOPE = 0.2


def _iota16():
    return lax.broadcasted_iota(jnp.int32, (16,), 0)


def _store1(ref, idxs, val, dtype):
    lane0 = _iota16() == 0
    vecs = [jnp.full((16,), i, jnp.int32) for i in idxs]
    plsc.store_scatter(ref, vecs, jnp.full((16,), val, dtype), mask=lane0)


def _add1(ref, idxs, val, dtype):
    lane0 = _iota16() == 0
    vecs = [jnp.full((16,), i, jnp.int32) for i in idxs]
    plsc.addupdate_scatter(ref, vecs, jnp.full((16,), val, dtype), mask=lane0)


def _load1(ref, idxs):
    lane0 = _iota16() == 0
    vecs = [jnp.full((16,), i, jnp.int32) for i in idxs]
    return plsc.load_gather(ref, vecs, mask=lane0)[0]


def _row(v):
    return lax.shift_right_logical(v, 7)


def _col(v):
    return v & 127


def _k1_body(dst_hbm, src_hbm, deg_hbm, e2s_hbm, e2d_hbm, e2c_hbm,
             dst_v, src_v, degloc_v, acc_v, tmp_v, e2src_v, e2dst_v, cnt16_v,
             sh_all, sem):
    cid = lax.axis_index("c")
    sid = lax.axis_index("s")
    wid = cid * 16 + sid
    epw = N_EDGES // 32

    def zero_body(j, _):
        for c in range(8):
            degloc_v[j, pl.ds(c * 16, 16)] = jnp.zeros((16,), jnp.float32)
        return 0
    lax.fori_loop(0, NR, zero_body, 0)

    pltpu.sync_copy(dst_hbm.at[pl.ds(wid * epw, epw)], dst_v)
    pltpu.sync_copy(src_hbm.at[pl.ds(wid * epw, epw)], src_v)

    def hist_body(i, off):
        v = dst_v[pl.ds(i * 16, 16)]
        cnt, lastm = plsc.scan_count(v)
        plsc.addupdate_scatter(
            degloc_v, [_row(v), _col(v)], cnt.astype(jnp.float32), mask=lastm)
        m = v >= NODE_A
        sv = src_v[pl.ds(i * 16, 16)]
        plsc.store_compressed(e2src_v.at[pl.ds(off, 16)], sv, mask=m)
        plsc.store_compressed(e2dst_v.at[pl.ds(off, 16)], v, mask=m)
        off = off + jnp.sum(m.astype(jnp.int32))
        return jnp.minimum(off, E2CAP - 16)
    off2 = lax.fori_loop(0, epw // 16, hist_body, jnp.int32(0))
    cnt16_v[...] = jnp.full((16,), off2, jnp.int32)
    pltpu.sync_copy(e2src_v, e2s_hbm.at[wid])
    pltpu.sync_copy(e2dst_v, e2d_hbm.at[wid])
    pltpu.sync_copy(cnt16_v, e2c_hbm.at[wid])

    pltpu.sync_copy(degloc_v, sh_all.at[sid])
    plsc.subcore_barrier()

    rpt = NR // 16
    base = sid * rpt
    pltpu.sync_copy(sh_all.at[0, pl.ds(base, rpt)], acc_v)
    for k in range(1, 16):
        pltpu.sync_copy(sh_all.at[k, pl.ds(base, rpt)], tmp_v)

        def add_body(j, _):
            for c in range(8):
                acc_v[j, pl.ds(c * 16, 16)] = (
                    acc_v[j, pl.ds(c * 16, 16)] + tmp_v[j, pl.ds(c * 16, 16)])
            return 0
        lax.fori_loop(0, rpt, add_body, 0)

    pltpu.sync_copy(acc_v, deg_hbm.at[cid, pl.ds(base, rpt)])


def _make_k1():
    mesh = plsc.VectorSubcoreMesh(core_axis_name="c", subcore_axis_name="s")
    return pl.kernel(
        _k1_body,
        out_type=(
            jax.ShapeDtypeStruct((2, NR, 128), jnp.float32),
            jax.ShapeDtypeStruct((32, E2CAP), jnp.int32),
            jax.ShapeDtypeStruct((32, E2CAP), jnp.int32),
            jax.ShapeDtypeStruct((32, 16), jnp.int32),
        ),
        mesh=mesh,
        compiler_params=pltpu.CompilerParams(needs_layout_passes=False),
        scratch_types=[
            pltpu.VMEM((N_EDGES // 32,), jnp.int32),
            pltpu.VMEM((N_EDGES // 32,), jnp.int32),
            pltpu.VMEM((NR, 128), jnp.float32),
            pltpu.VMEM((NR // 16, 128), jnp.float32),
            pltpu.VMEM((NR // 16, 128), jnp.float32),
            pltpu.VMEM((E2CAP,), jnp.int32),
            pltpu.VMEM((E2CAP,), jnp.int32),
            pltpu.VMEM((16,), jnp.int32),
            pltpu.VMEM_SHARED((16, NR, 128), jnp.float32),
            pltpu.SemaphoreType.DMA,
        ],
    )


def _k2_body(src_hbm, dst_hbm, dinv_hbm, e2s_hbm, e2d_hbm, e2c_hbm, x_hbm,
             agg1_hbm, amat_hbm,
             src_v, dst_v, dinv_v, table_v,
             cnt16_v,
             e1src_v, e1dst_v, e1slot_v, e1w_v,
             slotnodes_v, amat_v,
             e2a_src_v, e2a_dst_v, e2a_cnt_v,
             chunk_src_v, chunk_src2_v, rows_v, rows2_v, tmp16_v, agg1loc_v,
             sh_slotnodes, sh_cnt,
             sh_slab,
             sem, sem2):
    sid = lax.axis_index("s")
    epw = N_EDGES // 16

    pltpu.sync_copy(dinv_hbm, dinv_v)

    def zrow_body(j, _):
        for c in range(8):
            agg1loc_v[j, pl.ds(c * 16, 16)] = jnp.zeros((16,), jnp.float32)
        return 0
    lax.fori_loop(0, CS, zrow_body, 0)

    pltpu.sync_copy(dst_hbm.at[pl.ds(sid * epw, epw)], dst_v)
    pltpu.sync_copy(src_hbm.at[pl.ds(sid * epw, epw)], src_v)

    @pl.when(sid == 0)
    def _dedup():
        def tneg_body(j, _):
            for c in range(8):
                table_v[j, pl.ds(c * 16, 16)] = jnp.full((16,), -1, jnp.int32)
            return 0
        lax.fori_loop(0, NR, tneg_body, 0)

        def sn_body(j, _):
            slotnodes_v[pl.ds(j * 16, 16)] = jnp.zeros((16,), jnp.int32)
            return 0
        lax.fori_loop(0, CS // 16, sn_body, 0)

        def az_body(j, _):
            for r in range(8):
                amat_v[r, pl.ds(j * 16, 16)] = jnp.zeros((16,), jnp.float32)
            return 0
        lax.fori_loop(0, CS // 16, az_body, 0)

        pltpu.sync_copy(e2s_hbm, e2a_src_v)
        pltpu.sync_copy(e2d_hbm, e2a_dst_v)
        pltpu.sync_copy(e2c_hbm, e2a_cnt_v)

        _store1(table_v, [NODE_A >> 7, NODE_A & 127], jnp.int32(0), jnp.int32)
        _store1(table_v, [NODE_B >> 7, NODE_B & 127], jnp.int32(1), jnp.int32)
        _store1(slotnodes_v, [0], jnp.int32(NODE_A), jnp.int32)
        _store1(slotnodes_v, [1], jnp.int32(NODE_B), jnp.int32)
        dA = _load1(dinv_v, [NODE_A >> 7, NODE_A & 127])
        dB = _load1(dinv_v, [NODE_B >> 7, NODE_B & 127])
        _store1(amat_v, [0, 0], dA * dA, jnp.float32)
        _store1(amat_v, [1, 1], dB * dB, jnp.float32)

        cnt = jnp.int32(2)
        for t in range(32):
            ct = e2a_cnt_v[t, pl.ds(0, 16)][0]

            def e2e_body(i, cnt):
                s = _load1(e2a_src_v, [t, i])
                d = _load1(e2a_dst_v, [t, i])
                sl = _load1(table_v, [_row(s), _col(s)])
                isnew = sl < 0
                slot = jnp.where(isnew, cnt, sl)
                slot = jnp.minimum(slot, CS - 1)
                _store1(table_v, [_row(s), _col(s)], slot, jnp.int32)
                _store1(slotnodes_v, [slot], s, jnp.int32)
                w = (_load1(dinv_v, [_row(s), _col(s)])
                     * _load1(dinv_v, [_row(d), _col(d)]))
                r = d - NODE_A
                _add1(amat_v, [r, slot], w, jnp.float32)
                return cnt + isnew.astype(jnp.int32)
            cnt = lax.fori_loop(0, ct, e2e_body, cnt)

        cnt = jnp.minimum(cnt, CS)
        pltpu.sync_copy(amat_v, amat_hbm)
        pltpu.sync_copy(slotnodes_v, sh_slotnodes)
        cnt16_v[...] = jnp.full((16,), cnt, jnp.int32)
        pltpu.sync_copy(cnt16_v, sh_cnt)

    plsc.subcore_barrier()

    pltpu.sync_copy(sh_slotnodes, slotnodes_v)
    pltpu.sync_copy(sh_cnt, cnt16_v)
    cnt = cnt16_v[...][0]

    @pl.when(sid != 0)
    def _rebuild():
        def tneg_body(j, _):
            for c in range(8):
                table_v[j, pl.ds(c * 16, 16)] = jnp.full((16,), -1, jnp.int32)
            return 0
        lax.fori_loop(0, NR, tneg_body, 0)

        def tb_cond(j):
            return j < cnt

        def tb_body(j):
            n = _load1(slotnodes_v, [j])
            _store1(table_v, [_row(n), _col(n)], j, jnp.int32)
            return j + 1
        lax.while_loop(tb_cond, tb_body, jnp.int32(0))

    def ez_body(j, _):
        z = jnp.zeros((16,), jnp.int32)
        e1src_v[pl.ds(j * 16, 16)] = z
        e1dst_v[pl.ds(j * 16, 16)] = z
        e1slot_v[pl.ds(j * 16, 16)] = z
        return 0
    lax.fori_loop(0, E1CAP // 16, ez_body, 0)

    def e1_body(i, off):
        for u in range(2):
            v = dst_v[pl.ds((2 * i + u) * 16, 16)]
            tv = plsc.load_gather(table_v, [_row(v), _col(v)])
            m = tv >= 0
            sv = src_v[pl.ds((2 * i + u) * 16, 16)]
            plsc.store_compressed(e1src_v.at[pl.ds(off, 16)], sv, mask=m)
            plsc.store_compressed(e1dst_v.at[pl.ds(off, 16)], v, mask=m)
            plsc.store_compressed(e1slot_v.at[pl.ds(off, 16)], tv, mask=m)
            off = off + jnp.sum(m.astype(jnp.int32))
            off = jnp.minimum(off, E1CAP - 16)
        return off
    off1 = lax.fori_loop(0, epw // 32, e1_body, jnp.int32(0))

    def self_cond(state):
        j, _ = state
        return j < cnt

    def self_body(state):
        j, off = state
        n = _load1(slotnodes_v, [j])
        _store1(e1src_v, [off], n, jnp.int32)
        _store1(e1dst_v, [off], n, jnp.int32)
        _store1(e1slot_v, [off], j, jnp.int32)
        return (j + 16, jnp.minimum(off + 1, E1CAP - 1))
    _, off1 = lax.while_loop(self_cond, self_body, (sid, off1))
    nloc = off1

    def w_body(k, _):
        s = e1src_v[pl.ds(k * 16, 16)]
        d = e1dst_v[pl.ds(k * 16, 16)]
        wv = (plsc.load_gather(dinv_v, [_row(s), _col(s)])
              * plsc.load_gather(dinv_v, [_row(d), _col(d)]))
        lane = k * 16 + _iota16()
        e1w_v[pl.ds(k * 16, 16)] = jnp.where(lane < nloc, wv, 0.0)
        return 0
    lax.fori_loop(0, E1CAP // 16, w_body, 0)

    nch = lax.shift_right_logical(nloc + 15, 4)

    def issue(k, idxbuf, buf, bsem):
        idxbuf[...] = e1src_v[pl.ds(k * 16, 16)]
        pltpu.async_copy(x_hbm.at[idxbuf], buf, bsem)

    @pl.when(nch > 0)
    def _prime():
        issue(0, chunk_src_v, rows_v, sem)

    def chunk_body(k, _):
        parity = lax.rem(k, 2)

        @pl.when(k < nch)
        def _do():
            @pl.when(k + 1 < nch)
            def _next():
                @pl.when(parity == 0)
                def _n0():
                    issue(k + 1, chunk_src2_v, rows2_v, sem2)

                @pl.when(parity == 1)
                def _n1():
                    issue(k + 1, chunk_src_v, rows_v, sem)

            wchunk = e1w_v[pl.ds(k * 16, 16)]
            slotchunk = e1slot_v[pl.ds(k * 16, 16)]

            def accum(idxbuf, buf, bsem):
                pltpu.make_async_copy(x_hbm.at[idxbuf], buf, bsem).wait()
                for r in range(16):
                    wb = jnp.full((16,), wchunk[r], jnp.float32)
                    slot = slotchunk[r]
                    for c in range(8):
                        agg1loc_v[slot, pl.ds(c * 16, 16)] = (
                            agg1loc_v[slot, pl.ds(c * 16, 16)]
                            + buf[r, pl.ds(c * 16, 16)] * wb)

            @pl.when(parity == 0)
            def _a0():
                accum(chunk_src_v, rows_v, sem)

            @pl.when(parity == 1)
            def _a1():
                accum(chunk_src2_v, rows2_v, sem2)
        return 0
    lax.fori_loop(0, E1CAP // 16, chunk_body, 0)

    spt = CS // 16
    rb = sid * spt

    def zrow16(j, _):
        for c in range(8):
            rows_v[j, pl.ds(c * 16, 16)] = jnp.zeros((16,), jnp.float32)
        return 0
    lax.fori_loop(0, spt, zrow16, 0)

    for g in range(16):
        owner = lax.rem(sid + g, 16)
        obase = pl.multiple_of(owner * spt, spt)

        @pl.when(obase < cnt)
        def _pub():
            pltpu.sync_copy(agg1loc_v.at[pl.ds(obase, spt)], sh_slab.at[sid])
        plsc.subcore_barrier()

        @pl.when(rb < cnt)
        def _red():
            srcslot = lax.rem(sid - g + 16, 16)
            pltpu.sync_copy(sh_slab.at[srcslot], tmp16_v)

            def red_body(j, _):
                for c in range(8):
                    rows_v[j, pl.ds(c * 16, 16)] = (
                        rows_v[j, pl.ds(c * 16, 16)]
                        + tmp16_v[j, pl.ds(c * 16, 16)])
                return 0
            lax.fori_loop(0, spt, red_body, 0)
        plsc.subcore_barrier()

    pltpu.sync_copy(rows_v, agg1_hbm.at[pl.ds(rb, spt)])


def _make_k2():
    mesh = plsc.VectorSubcoreMesh(core_axis_name="c", subcore_axis_name="s",
                                  num_cores=1)
    epw = N_EDGES // 16
    return pl.kernel(
        _k2_body,
        out_type=(
            jax.ShapeDtypeStruct((CS, D_FEAT), jnp.float32),
            jax.ShapeDtypeStruct((8, CS), jnp.float32),
        ),
        mesh=mesh,
        compiler_params=pltpu.CompilerParams(needs_layout_passes=False),
        scratch_types=[
            pltpu.VMEM((epw,), jnp.int32),
            pltpu.VMEM((epw,), jnp.int32),
            pltpu.VMEM((NR, 128), jnp.float32),
            pltpu.VMEM((NR, 128), jnp.int32),
            pltpu.VMEM((16,), jnp.int32),
            pltpu.VMEM((E1CAP,), jnp.int32),
            pltpu.VMEM((E1CAP,), jnp.int32),
            pltpu.VMEM((E1CAP,), jnp.int32),
            pltpu.VMEM((E1CAP,), jnp.float32),
            pltpu.VMEM((CS,), jnp.int32),
            pltpu.VMEM((8, CS), jnp.float32),
            pltpu.VMEM((32, E2CAP), jnp.int32),
            pltpu.VMEM((32, E2CAP), jnp.int32),
            pltpu.VMEM((32, 16), jnp.int32),
            pltpu.VMEM((16,), jnp.int32),
            pltpu.VMEM((16,), jnp.int32),
            pltpu.VMEM((16, D_FEAT), jnp.float32),
            pltpu.VMEM((16, D_FEAT), jnp.float32),
            pltpu.VMEM((16, D_FEAT), jnp.float32),
            pltpu.VMEM((CS, D_FEAT), jnp.float32),
            pltpu.VMEM_SHARED((CS,), jnp.int32),
            pltpu.VMEM_SHARED((16,), jnp.int32),
            pltpu.VMEM_SHARED((16, CS // 16, D_FEAT), jnp.float32),
            pltpu.SemaphoreType.DMA,
            pltpu.SemaphoreType.DMA,
        ],
    )


def _k1b_body(degp_ref, dinv_ref):
    d = degp_ref[0] + degp_ref[1] + 1.0
    dinv_ref[...] = lax.rsqrt(d)


def _k1b(deg_parts):
    return pl.pallas_call(
        _k1b_body,
        out_shape=jax.ShapeDtypeStruct((NR, 128), jnp.float32),
    )(deg_parts)


def _k3_body(agg1_ref, w1_ref, b1_ref, amat_ref, w2_ref, b2_ref, out_ref):
    h1 = jnp.dot(agg1_ref[...], w1_ref[...],
                 preferred_element_type=jnp.float32) + b1_ref[...]
    act = jnp.where(h1 >= 0, h1, h1 * _NEG_SLOPE)
    t = jnp.dot(amat_ref[...], act, preferred_element_type=jnp.float32)
    h2 = jnp.dot(t, w2_ref[...],
                 preferred_element_type=jnp.float32) + b2_ref[...]
    col = lax.broadcasted_iota(jnp.int32, (1, 128), 1)
    valid = col < 16
    rowb = h2[1:2, :]
    m = jnp.max(jnp.where(valid, rowb, -jnp.inf))
    e = jnp.where(valid, jnp.exp(rowb - m), 0.0)
    f1 = e / jnp.sum(e)
    rowa = h2[0:1, :]
    out_ref[...] = jnp.concatenate(
        [f1, rowa, jnp.zeros((6, 128), jnp.float32)], axis=0)


def _k3(agg1, w1p, b1p, amat, w2p, b2p):
    return pl.pallas_call(
        _k3_body,
        out_shape=jax.ShapeDtypeStruct((8, 128), jnp.float32),
    )(agg1, w1p, b1p, amat, w2p, b2p)


@jax.jit
def kernel(x, edge_index, W1, b1, W2, b2):
    src = edge_index[0].astype(jnp.int32)
    dst = edge_index[1].astype(jnp.int32)

    deg_parts, e2s, e2d, e2c = _make_k1()(dst, src)
    dinv = _k1b(deg_parts)
    agg1, amat = _make_k2()(src, dst, dinv, e2s, e2d, e2c, x)

    hidden = W1.shape[1]
    w1p = jnp.pad(W1, ((0, 0), (0, 256 - hidden)))
    b1p = jnp.pad(b1, (0, 256 - hidden)).reshape(1, 256)
    w2p = jnp.pad(W2, ((0, 256 - hidden), (0, 128 - W2.shape[1])))
    b2p = jnp.pad(b2, (0, 128 - W2.shape[1])).reshape(1, 128)

    out = _k3(agg1, w1p, b1p, amat, w2p, b2p)
    return jnp.concatenate([out[0, :16], out[1, :16]], axis=0)
